# Initial kernel scaffold; baseline (speedup 1.0000x reference)
import jax
import jax.numpy as jnp
from jax.experimental import pallas as pl


def kernel(z, W1, b1, W2, b2, W3, b3, a_raw, b_out):
    raise NotImplementedError("write your pallas kernel here")


# speedup vs baseline: 2.6415x; 2.6415x over previous
import functools

import jax
import jax.numpy as jnp
from jax import lax
from jax.experimental import pallas as pl
from jax.experimental.pallas import tpu as pltpu, tpu_sc as plsc

N_POINTS = 4096
HIDDEN = 1024
Z_MIN, Z_MAX = -3.0, 3.0
C = 1.0
DT = (Z_MAX - Z_MIN) / (N_POINTS - 1)
INV_DT = 1.0 / DT

ROWS = 512
GRID = N_POINTS // ROWS

NB = 16384
NW = 32
CHUNK = NB // NW
LANES = 16

_PREC = lax.Precision.HIGHEST


def _mlp_body(W1_ref, b1_ref, W2_ref, b2_ref, W3_ref, b3_ref, w_ref):
    i = pl.program_id(0)
    row = (lax.broadcasted_iota(jnp.int32, (ROWS, 1), 0) + i * ROWS).astype(jnp.float32)
    t = Z_MIN + row * DT
    h = jnp.tanh(t * W1_ref[...] + b1_ref[...])
    h = jnp.tanh(
        lax.dot_general(h, W2_ref[...], (((1,), (1,)), ((), ())),
                        preferred_element_type=jnp.float32, precision=_PREC)
        + b2_ref[...])
    g8 = lax.dot_general(h, W3_ref[...], (((1,), (1,)), ((), ())),
                         preferred_element_type=jnp.float32, precision=_PREC)
    g = g8[:, 0:1] + b3_ref[0, 0]
    w_ref[...] = jnp.exp(jnp.clip(g, -C, C))


def _cumsum_body(w_ref, F_ref):
    X = w_ref[...]
    k = lax.broadcasted_iota(jnp.int32, (128, 128), 0)
    j = lax.broadcasted_iota(jnp.int32, (128, 128), 1)
    U = (k < j).astype(jnp.float32)
    row_excl = lax.dot_general(X, U, (((1,), (0,)), ((), ())),
                               preferred_element_type=jnp.float32, precision=_PREC)
    r = jnp.broadcast_to(jnp.sum(X, axis=1, keepdims=True), (32, 8))
    ii = lax.broadcasted_iota(jnp.int32, (32, 32), 0)
    mm = lax.broadcasted_iota(jnp.int32, (32, 32), 1)
    V = (mm < ii).astype(jnp.float32)
    roffs = lax.dot_general(V, r, (((1,), (0,)), ((), ())),
                            preferred_element_type=jnp.float32, precision=_PREC)
    E = row_excl + roffs[:, 0:1]
    w00 = X[0, 0]
    F_ref[...] = DT * E + (0.5 * DT) * (X - w00)


_sc_mesh = plsc.VectorSubcoreMesh(core_axis_name="c", subcore_axis_name="s")


@functools.partial(
    pl.kernel,
    mesh=_sc_mesh,
    compiler_params=pltpu.CompilerParams(needs_layout_passes=False),
    out_type=jax.ShapeDtypeStruct((NB,), jnp.float32),
    scratch_types=[
        pltpu.VMEM((N_POINTS,), jnp.float32),
        pltpu.VMEM((N_POINTS,), jnp.float32),
        pltpu.VMEM((CHUNK,), jnp.float32),
        pltpu.VMEM((CHUNK,), jnp.float32),
    ],
)
def _gather_lerp(F_hbm, w_hbm, z_hbm, out_hbm, F_v, w_v, z_v, o_v):
    wid = lax.axis_index("s") * 2 + lax.axis_index("c")
    base = wid * CHUNK
    pltpu.sync_copy(F_hbm, F_v)
    pltpu.sync_copy(w_hbm, w_v)
    pltpu.sync_copy(z_hbm.at[pl.ds(base, CHUNK)], z_v)
    w_head = w_v[pl.ds(0, LANES)]
    w_tail = w_v[pl.ds(N_POINTS - LANES, LANES)]
    F_tail = F_v[pl.ds(N_POINTS - LANES, LANES)]
    w0 = w_head[0]
    wN = w_tail[LANES - 1]
    FN = F_tail[LANES - 1]

    def body(i, carry):
        zv = z_v[pl.ds(i * LANES, LANES)]
        pos = (zv - Z_MIN) * INV_DT
        idx = jnp.clip(pos.astype(jnp.int32), 0, N_POINTS - 2)
        frac = jnp.clip(pos - idx.astype(jnp.float32), 0.0, 1.0)
        F_lo = plsc.load_gather(F_v, [idx])
        w_lo = plsc.load_gather(w_v, [idx])
        F_mid = F_lo + frac * w_lo * DT
        F_low = (zv - Z_MIN) * w0
        F_high = FN + (zv - Z_MAX) * wN
        fz = jnp.where(zv < Z_MIN, F_low, jnp.where(zv > Z_MAX, F_high, F_mid))
        o_v[pl.ds(i * LANES, LANES)] = fz
        return carry

    lax.fori_loop(0, CHUNK // LANES, body, 0)
    pltpu.sync_copy(o_v, out_hbm.at[pl.ds(base, CHUNK)])


def _norm_body(Fz_ref, a_ref, b_ref, out_ref):
    X = Fz_ref[...]
    mu = jnp.sum(X) * (1.0 / NB)
    d = X - mu
    var = jnp.sum(d * d) * (1.0 / (NB - 1))
    sigma = jnp.maximum(jnp.sqrt(var), 0.001)
    ar = a_ref[0, 0]
    a = jnp.maximum(ar, 0.0) + jnp.log(1.0 + jnp.exp(-jnp.abs(ar))) + 0.001
    out_ref[...] = (a / (sigma + 1e-6)) * d + b_ref[0, 0]


def kernel(z, W1, b1, W2, b2, W3, b3, a_raw, b_out):
    H = HIDDEN
    W1r = W1.reshape(1, H)
    b1r = b1.reshape(1, H)
    b2r = b2.reshape(1, H)
    W3r = jnp.broadcast_to(W3.reshape(1, H), (8, H))
    b3r = b3.reshape(1, 1)

    full = lambda shape: pl.BlockSpec(shape, lambda i: (0, 0))
    w_col = pl.pallas_call(
        _mlp_body,
        grid=(GRID,),
        in_specs=[full((1, H)), full((1, H)), full((H, H)), full((1, H)),
                  full((8, H)), full((1, 1))],
        out_specs=pl.BlockSpec((ROWS, 1), lambda i: (i, 0)),
        out_shape=jax.ShapeDtypeStruct((N_POINTS, 1), jnp.float32),
    )(W1r, b1r, W2, b2r, W3r, b3r)

    w2d = w_col.reshape(32, 128)
    F2d = pl.pallas_call(
        _cumsum_body,
        out_shape=jax.ShapeDtypeStruct((32, 128), jnp.float32),
    )(w2d)

    Fz = _gather_lerp(F2d.reshape(N_POINTS), w_col.reshape(N_POINTS),
                      z.reshape(NB))

    out2d = pl.pallas_call(
        _norm_body,
        out_shape=jax.ShapeDtypeStruct((128, 128), jnp.float32),
    )(Fz.reshape(128, 128), a_raw.reshape(1, 1), b_out.reshape(1, 1))
    return out2d.reshape(z.shape)


# speedup vs baseline: 6.0393x; 2.2863x over previous
import functools

import jax
import jax.numpy as jnp
from jax import lax
from jax.experimental import pallas as pl
from jax.experimental.pallas import tpu as pltpu, tpu_sc as plsc

N_POINTS = 4096
HIDDEN = 1024
Z_MIN, Z_MAX = -3.0, 3.0
C = 1.0
DT = (Z_MAX - Z_MIN) / (N_POINTS - 1)
INV_DT = 1.0 / DT

ROWS = 512
GRID = N_POINTS // ROWS

NB = 16384
NW = 32
CHUNK = NB // NW
LANES = 16

_PREC = lax.Precision.HIGHEST
_MLP_PREC = lax.Precision.DEFAULT


def _mlp_body(W1_ref, b1_ref, W2_ref, b2_ref, W3_ref, b3_ref, w_ref):
    i = pl.program_id(0)
    row = (lax.broadcasted_iota(jnp.int32, (ROWS, 1), 0) + i * ROWS).astype(jnp.float32)
    t = Z_MIN + row * DT
    h = jnp.tanh(t * W1_ref[...] + b1_ref[...])
    h = jnp.tanh(
        lax.dot_general(h, W2_ref[...], (((1,), (1,)), ((), ())),
                        preferred_element_type=jnp.float32, precision=_MLP_PREC)
        + b2_ref[...])
    g8 = lax.dot_general(h, W3_ref[...], (((1,), (1,)), ((), ())),
                         preferred_element_type=jnp.float32, precision=_MLP_PREC)
    g = g8[:, 0:1] + b3_ref[0, 0]
    w_ref[...] = jnp.exp(jnp.clip(g, -C, C))


def _cumsum_body(w_ref, F_ref):
    X = w_ref[...]
    k = lax.broadcasted_iota(jnp.int32, (128, 128), 0)
    j = lax.broadcasted_iota(jnp.int32, (128, 128), 1)
    U = (k < j).astype(jnp.float32)
    row_excl = lax.dot_general(X, U, (((1,), (0,)), ((), ())),
                               preferred_element_type=jnp.float32, precision=_PREC)
    r = jnp.broadcast_to(jnp.sum(X, axis=1, keepdims=True), (32, 8))
    ii = lax.broadcasted_iota(jnp.int32, (32, 32), 0)
    mm = lax.broadcasted_iota(jnp.int32, (32, 32), 1)
    V = (mm < ii).astype(jnp.float32)
    roffs = lax.dot_general(V, r, (((1,), (0,)), ((), ())),
                            preferred_element_type=jnp.float32, precision=_PREC)
    E = row_excl + roffs[:, 0:1]
    w00 = X[0, 0]
    F_ref[...] = DT * E + (0.5 * DT) * (X - w00)


_sc_mesh = plsc.VectorSubcoreMesh(core_axis_name="c", subcore_axis_name="s")


@functools.partial(
    pl.kernel,
    mesh=_sc_mesh,
    compiler_params=pltpu.CompilerParams(needs_layout_passes=False),
    out_type=jax.ShapeDtypeStruct((NB,), jnp.float32),
    scratch_types=[
        pltpu.VMEM((N_POINTS,), jnp.float32),
        pltpu.VMEM((N_POINTS,), jnp.float32),
        pltpu.VMEM((CHUNK,), jnp.float32),
        pltpu.VMEM((CHUNK,), jnp.float32),
    ],
)
def _gather_lerp(F_hbm, w_hbm, z_hbm, out_hbm, F_v, w_v, z_v, o_v):
    wid = lax.axis_index("s") * 2 + lax.axis_index("c")
    base = wid * CHUNK
    pltpu.sync_copy(F_hbm, F_v)
    pltpu.sync_copy(w_hbm, w_v)
    pltpu.sync_copy(z_hbm.at[pl.ds(base, CHUNK)], z_v)
    w_head = w_v[pl.ds(0, LANES)]
    w_tail = w_v[pl.ds(N_POINTS - LANES, LANES)]
    F_tail = F_v[pl.ds(N_POINTS - LANES, LANES)]
    w0 = w_head[0]
    wN = w_tail[LANES - 1]
    FN = F_tail[LANES - 1]

    def body(i, carry):
        zv = z_v[pl.ds(i * LANES, LANES)]
        pos = (zv - Z_MIN) * INV_DT
        idx = jnp.clip(pos.astype(jnp.int32), 0, N_POINTS - 2)
        frac = jnp.clip(pos - idx.astype(jnp.float32), 0.0, 1.0)
        F_lo = plsc.load_gather(F_v, [idx])
        w_lo = plsc.load_gather(w_v, [idx])
        F_mid = F_lo + frac * w_lo * DT
        F_low = (zv - Z_MIN) * w0
        F_high = FN + (zv - Z_MAX) * wN
        fz = jnp.where(zv < Z_MIN, F_low, jnp.where(zv > Z_MAX, F_high, F_mid))
        o_v[pl.ds(i * LANES, LANES)] = fz
        return carry

    lax.fori_loop(0, CHUNK // LANES, body, 0)
    pltpu.sync_copy(o_v, out_hbm.at[pl.ds(base, CHUNK)])


def _norm_body(Fz_ref, a_ref, b_ref, out_ref):
    X = Fz_ref[...]
    mu = jnp.sum(X) * (1.0 / NB)
    d = X - mu
    var = jnp.sum(d * d) * (1.0 / (NB - 1))
    sigma = jnp.maximum(jnp.sqrt(var), 0.001)
    ar = a_ref[0, 0]
    a = jnp.maximum(ar, 0.0) + jnp.log(1.0 + jnp.exp(-jnp.abs(ar))) + 0.001
    out_ref[...] = (a / (sigma + 1e-6)) * d + b_ref[0, 0]


def kernel(z, W1, b1, W2, b2, W3, b3, a_raw, b_out):
    H = HIDDEN
    W1r = W1.reshape(1, H)
    b1r = b1.reshape(1, H)
    b2r = b2.reshape(1, H)
    W3r = jnp.broadcast_to(W3.reshape(1, H), (8, H))
    b3r = b3.reshape(1, 1)

    full = lambda shape: pl.BlockSpec(shape, lambda i: (0, 0))
    w_col = pl.pallas_call(
        _mlp_body,
        grid=(GRID,),
        in_specs=[full((1, H)), full((1, H)), full((H, H)), full((1, H)),
                  full((8, H)), full((1, 1))],
        out_specs=pl.BlockSpec((ROWS, 1), lambda i: (i, 0)),
        out_shape=jax.ShapeDtypeStruct((N_POINTS, 1), jnp.float32),
    )(W1r, b1r, W2, b2r, W3r, b3r)

    w2d = w_col.reshape(32, 128)
    F2d = pl.pallas_call(
        _cumsum_body,
        out_shape=jax.ShapeDtypeStruct((32, 128), jnp.float32),
    )(w2d)

    Fz = _gather_lerp(F2d.reshape(N_POINTS), w_col.reshape(N_POINTS),
                      z.reshape(NB))

    out2d = pl.pallas_call(
        _norm_body,
        out_shape=jax.ShapeDtypeStruct((128, 128), jnp.float32),
    )(Fz.reshape(128, 128), a_raw.reshape(1, 1), b_out.reshape(1, 1))
    return out2d.reshape(z.shape)
